# baseline (speedup 1.0000x reference)
import jax
import jax.numpy as jnp
from jax import lax
from jax.experimental import pallas as pl
from jax.experimental.pallas import tpu as pltpu
from jax.experimental.pallas import tpu_sc as plsc

PROMPT_NUM = 50
HIDDEN_SIZE = 1024
BATCH = 1024

_PP = 2
_NSTEP = PROMPT_NUM // _PP
_NBUF = 4

_NW = 32
_MASK_PER_W = BATCH * PROMPT_NUM // _NW


def _tc_body(table_v, emb_hbm, staged, sems):
    handles = []
    for t in range(_NSTEP):
        s = t % _NBUF
        if t >= _NBUF:
            handles[t - _NBUF].wait()
        staged[s, ...] = jnp.broadcast_to(
            table_v[pl.ds(t * _PP, _PP), :][:, None, :],
            (_PP, BATCH, HIDDEN_SIZE),
        )
        h = pltpu.make_async_copy(
            staged.at[s], emb_hbm.at[pl.ds(t * _PP, _PP)], sems.at[s]
        )
        h.start()
        handles.append(h)
    for t in range(_NSTEP - _NBUF, _NSTEP):
        handles[t].wait()


def _tc_broadcast(prompt_table):
    return pl.pallas_call(
        _tc_body,
        out_shape=jax.ShapeDtypeStruct(
            (PROMPT_NUM, BATCH, HIDDEN_SIZE), jnp.float32
        ),
        in_specs=[pl.BlockSpec(memory_space=pltpu.VMEM)],
        out_specs=pl.BlockSpec(memory_space=pl.ANY),
        scratch_shapes=[
            pltpu.VMEM((_NBUF, _PP, BATCH, HIDDEN_SIZE), jnp.float32),
            pltpu.SemaphoreType.DMA((_NBUF,)),
        ],
    )(prompt_table)


def _sc_mask_body(mask_hbm, ones_v):
    wid = lax.axis_index("s") * 2 + lax.axis_index("c")

    def _fill(i, carry):
        ones_v[pl.ds(i * 16, 16)] = jnp.ones((16,), jnp.float32)
        return carry

    lax.fori_loop(0, _MASK_PER_W // 16, _fill, 0)
    pltpu.sync_copy(ones_v, mask_hbm.at[pl.ds(wid * _MASK_PER_W, _MASK_PER_W)])


def _sc_mask():
    mesh = plsc.VectorSubcoreMesh(core_axis_name="c", subcore_axis_name="s")
    return pl.kernel(
        _sc_mask_body,
        out_type=jax.ShapeDtypeStruct((BATCH * PROMPT_NUM,), jnp.float32),
        mesh=mesh,
        scratch_types=[pltpu.VMEM((_MASK_PER_W,), jnp.float32)],
    )()


def kernel(batch_size, prompt_table):
    emb_t = _tc_broadcast(prompt_table)
    emb = jnp.transpose(emb_t, (1, 0, 2))
    mask = _sc_mask().reshape(BATCH, PROMPT_NUM)
    return emb, mask


# speedup vs baseline: 1.2690x; 1.2690x over previous
import jax
import jax.numpy as jnp
from jax import lax
from jax.experimental import pallas as pl
from jax.experimental.pallas import tpu as pltpu
from jax.experimental.pallas import tpu_sc as plsc

PROMPT_NUM = 50
HIDDEN_SIZE = 1024
BATCH = 1024

_PP = 2
_NSTEP = PROMPT_NUM // _PP
_NBUF = 4


def _tc_body(table_v, emb_hbm, mask_hbm, staged, ones_v, sems, mask_sem):
    handles = []
    mask_started = False
    for t in range(_NSTEP):
        s = t % _NBUF
        if t >= _NBUF:
            handles[t - _NBUF].wait()
        staged[s, ...] = jnp.broadcast_to(
            table_v[pl.ds(t * _PP, _PP), :][:, None, :],
            (_PP, BATCH, HIDDEN_SIZE),
        )
        h = pltpu.make_async_copy(
            staged.at[s], emb_hbm.at[pl.ds(t * _PP, _PP)], sems.at[s]
        )
        h.start()
        handles.append(h)
        if not mask_started:
            ones_v[...] = jnp.ones((PROMPT_NUM, BATCH), jnp.float32)
            mask_started = True
    mask_h = pltpu.make_async_copy(ones_v, mask_hbm, mask_sem)
    mask_h.start()
    for t in range(_NSTEP - _NBUF, _NSTEP):
        handles[t].wait()
    mask_h.wait()


def _tc_broadcast(prompt_table):
    return pl.pallas_call(
        _tc_body,
        out_shape=(
            jax.ShapeDtypeStruct((PROMPT_NUM, BATCH, HIDDEN_SIZE), jnp.float32),
            jax.ShapeDtypeStruct((PROMPT_NUM, BATCH), jnp.float32),
        ),
        in_specs=[pl.BlockSpec(memory_space=pltpu.VMEM)],
        out_specs=(
            pl.BlockSpec(memory_space=pl.ANY),
            pl.BlockSpec(memory_space=pl.ANY),
        ),
        scratch_shapes=[
            pltpu.VMEM((_NBUF, _PP, BATCH, HIDDEN_SIZE), jnp.float32),
            pltpu.VMEM((PROMPT_NUM, BATCH), jnp.float32),
            pltpu.SemaphoreType.DMA((_NBUF,)),
            pltpu.SemaphoreType.DMA,
        ],
    )(prompt_table)


def kernel(batch_size, prompt_table):
    emb_t, mask_t = _tc_broadcast(prompt_table)
    emb = jnp.transpose(emb_t, (1, 0, 2))
    mask = jnp.transpose(mask_t, (1, 0))
    return emb, mask


# speedup vs baseline: 1.2793x; 1.0081x over previous
import jax
import jax.numpy as jnp
from jax import lax
from jax.experimental import pallas as pl
from jax.experimental.pallas import tpu as pltpu
from jax.experimental.pallas import tpu_sc as plsc

PROMPT_NUM = 50
HIDDEN_SIZE = 1024
BATCH = 1024

_PP = 5
_NSTEP = PROMPT_NUM // _PP
_NBUF = 2


def _tc_body(table_v, emb_hbm, mask_hbm, staged, ones_v, sems, mask_sem):
    handles = []
    mask_started = False
    for t in range(_NSTEP):
        s = t % _NBUF
        if t >= _NBUF:
            handles[t - _NBUF].wait()
        staged[s, ...] = jnp.broadcast_to(
            table_v[pl.ds(t * _PP, _PP), :][:, None, :],
            (_PP, BATCH, HIDDEN_SIZE),
        )
        h = pltpu.make_async_copy(
            staged.at[s], emb_hbm.at[pl.ds(t * _PP, _PP)], sems.at[s]
        )
        h.start()
        handles.append(h)
        if not mask_started:
            ones_v[...] = jnp.ones((PROMPT_NUM, BATCH), jnp.float32)
            mask_started = True
    mask_h = pltpu.make_async_copy(ones_v, mask_hbm, mask_sem)
    mask_h.start()
    for t in range(_NSTEP - _NBUF, _NSTEP):
        handles[t].wait()
    mask_h.wait()


def _tc_broadcast(prompt_table):
    return pl.pallas_call(
        _tc_body,
        out_shape=(
            jax.ShapeDtypeStruct((PROMPT_NUM, BATCH, HIDDEN_SIZE), jnp.float32),
            jax.ShapeDtypeStruct((PROMPT_NUM, BATCH), jnp.float32),
        ),
        in_specs=[pl.BlockSpec(memory_space=pltpu.VMEM)],
        out_specs=(
            pl.BlockSpec(memory_space=pl.ANY),
            pl.BlockSpec(memory_space=pl.ANY),
        ),
        scratch_shapes=[
            pltpu.VMEM((_NBUF, _PP, BATCH, HIDDEN_SIZE), jnp.float32),
            pltpu.VMEM((PROMPT_NUM, BATCH), jnp.float32),
            pltpu.SemaphoreType.DMA((_NBUF,)),
            pltpu.SemaphoreType.DMA,
        ],
    )(prompt_table)


def kernel(batch_size, prompt_table):
    emb_t, mask_t = _tc_broadcast(prompt_table)
    emb = jnp.transpose(emb_t, (1, 0, 2))
    mask = jnp.transpose(mask_t, (1, 0))
    return emb, mask


# speedup vs baseline: 1.2799x; 1.0005x over previous
import jax
import jax.numpy as jnp
from jax.experimental import pallas as pl
from jax.experimental.pallas import tpu as pltpu

PROMPT_NUM = 50
HIDDEN_SIZE = 1024
BATCH = 1024

_PP = 5
_NSTEP = PROMPT_NUM // _PP
_NBUF = 2


def _tc_body(table_v, emb_hbm, mask_hbm, staged, ones_v, sems, mask_sem):
    handles = []
    mask_started = False
    for t in range(_NSTEP):
        s = t % _NBUF
        if t >= _NBUF:
            handles[t - _NBUF].wait()
        staged[s, ...] = jnp.broadcast_to(
            table_v[pl.ds(t * _PP, _PP), :][:, None, :],
            (_PP, BATCH, HIDDEN_SIZE),
        )
        h = pltpu.make_async_copy(
            staged.at[s], emb_hbm.at[pl.ds(t * _PP, _PP)], sems.at[s]
        )
        h.start()
        handles.append(h)
        if not mask_started:
            ones_v[...] = jnp.ones((PROMPT_NUM, BATCH), jnp.float32)
            mask_started = True
    mask_h = pltpu.make_async_copy(ones_v, mask_hbm, mask_sem)
    mask_h.start()
    for t in range(_NSTEP - _NBUF, _NSTEP):
        handles[t].wait()
    mask_h.wait()


def _tc_broadcast(prompt_table):
    return pl.pallas_call(
        _tc_body,
        out_shape=(
            jax.ShapeDtypeStruct((PROMPT_NUM, BATCH, HIDDEN_SIZE), jnp.float32),
            jax.ShapeDtypeStruct((PROMPT_NUM, BATCH), jnp.float32),
        ),
        in_specs=[pl.BlockSpec(memory_space=pltpu.VMEM)],
        out_specs=(
            pl.BlockSpec(memory_space=pl.ANY),
            pl.BlockSpec(memory_space=pl.ANY),
        ),
        scratch_shapes=[
            pltpu.VMEM((_NBUF, _PP, BATCH, HIDDEN_SIZE), jnp.float32),
            pltpu.VMEM((PROMPT_NUM, BATCH), jnp.float32),
            pltpu.SemaphoreType.DMA((_NBUF,)),
            pltpu.SemaphoreType.DMA,
        ],
    )(prompt_table)


def kernel(batch_size, prompt_table):
    emb_t, mask_t = _tc_broadcast(prompt_table)
    emb = jnp.transpose(emb_t, (1, 0, 2))
    mask = jnp.transpose(mask_t, (1, 0))
    return emb, mask
